# Initial kernel scaffold; baseline (speedup 1.0000x reference)
import functools

import jax
import jax.numpy as jnp
from jax import lax
from jax.experimental import pallas as pl
from jax.experimental.pallas import tpu as pltpu
from jax.experimental.pallas import tpu_sc as plsc

N = 10000
E = 160000
D = 256
B = 16
HD = 128
NP = 10240
EP = 163840
NS = 16
EPT = EP // NS
CH = 512
NCH = EPT // CH
RPT = NP // NS
BLK = 1024

_F32 = jnp.float32
_HIGH = lax.Precision.HIGHEST


def _dot(a, b, dims=None):
    if dims is None:
        return jnp.dot(a, b, preferred_element_type=_F32, precision=_HIGH)
    return lax.dot_general(a, b, (dims, ((), ())),
                           preferred_element_type=_F32, precision=_HIGH)


def _sc_agg_kernel(h0_hbm, h1_hbm, src_hbm, dst_hbm, z_hbm, o0_hbm, o1_hbm,
                   sidx, didx, rows, acc, sem):
    cid = lax.axis_index("c")
    sid = lax.axis_index("s")

    pltpu.sync_copy(z_hbm.at[pl.ds(sid * RPT, RPT)],
                    acc.at[pl.ds(sid * RPT, RPT)])
    plsc.subcore_barrier()

    def run(h_hbm, out_hbm):
        def chunk(k, carry):
            rbase = sid * (EPT // 128) + k * (CH // 128)
            pltpu.sync_copy(src_hbm.at[pl.ds(rbase, CH // 128)], sidx)
            pltpu.sync_copy(dst_hbm.at[pl.ds(rbase, CH // 128)], didx)
            cps = [
                pltpu.async_copy(h_hbm.at[sidx.at[j]],
                                 rows.at[pl.ds(j * 128, 128)], sem)
                for j in range(CH // 128)
            ]
            for cp in cps:
                cp.wait()
            for j in range(CH // 128):
                pltpu.sync_copy(rows.at[pl.ds(j * 128, 128)],
                                acc.at[didx.at[j]], add=True)
            return carry

        lax.fori_loop(0, NCH, chunk, 0)
        plsc.subcore_barrier()
        pltpu.sync_copy(acc.at[pl.ds(sid * RPT, RPT)],
                        out_hbm.at[pl.ds(sid * RPT, RPT)])

    @pl.when(cid == 0)
    def _():
        run(h0_hbm, o0_hbm)

    @pl.when(cid == 1)
    def _():
        run(h1_hbm, o1_hbm)


def _sc_agg(h0, h1, src2d, dst2d, zeros):
    mesh = plsc.VectorSubcoreMesh(core_axis_name="c", subcore_axis_name="s")
    k = functools.partial(
        pl.kernel, mesh=mesh,
        out_type=[jax.ShapeDtypeStruct((NP, HD), _F32),
                  jax.ShapeDtypeStruct((NP, HD), _F32)],
        scratch_types=[
            pltpu.VMEM((CH // 128, 128), jnp.int32),
            pltpu.VMEM((CH // 128, 128), jnp.int32),
            pltpu.VMEM((CH, HD), _F32),
            pltpu.VMEM_SHARED((NP, HD), _F32),
            pltpu.SemaphoreType.DMA,
        ],
    )(_sc_agg_kernel)
    return k(h0, h1, src2d, dst2d, zeros)


def _round1_body(a0, a1, wa, wb, b, o0, o1):
    h = _dot(a0[...], wa[...]) + _dot(a1[...], wb[...]) + b[...]
    h = jnp.maximum(h, 0.0)
    o0[...] = h[:, :HD]
    o1[...] = h[:, HD:]


def _round1(a0, a1, wa, wb, b):
    return pl.pallas_call(
        _round1_body,
        grid=(NP // BLK,),
        in_specs=[
            pl.BlockSpec((BLK, HD), lambda i: (i, 0)),
            pl.BlockSpec((BLK, HD), lambda i: (i, 0)),
            pl.BlockSpec((HD, D), lambda i: (0, 0)),
            pl.BlockSpec((HD, D), lambda i: (0, 0)),
            pl.BlockSpec((1, D), lambda i: (0, 0)),
        ],
        out_specs=[pl.BlockSpec((BLK, HD), lambda i: (i, 0))] * 2,
        out_shape=[jax.ShapeDtypeStruct((NP, HD), _F32)] * 2,
    )(a0, a1, wa, wb, b)


def _round2_body(a0, a1, wa, wb, b, gpw, gpb, gid, h_ref, hg_ref):
    i = pl.program_id(0)
    h = _dot(a0[...], wa[...]) + _dot(a1[...], wb[...]) + b[...]
    h = jnp.maximum(h, 0.0)
    h_ref[...] = h
    y = _dot(h, gpw[...]) + gpb[...]
    iota = lax.broadcasted_iota(jnp.int32, (BLK, B), 1)
    mask = (gid[...] == iota).astype(_F32)
    part = _dot(mask, y, dims=((0,), (0,)))

    @pl.when(i == 0)
    def _():
        hg_ref[...] = part

    @pl.when(i > 0)
    def _():
        hg_ref[...] += part


def _round2(a0, a1, wa, wb, b, gpw, gpb, gid_col):
    return pl.pallas_call(
        _round2_body,
        grid=(NP // BLK,),
        in_specs=[
            pl.BlockSpec((BLK, HD), lambda i: (i, 0)),
            pl.BlockSpec((BLK, HD), lambda i: (i, 0)),
            pl.BlockSpec((HD, D), lambda i: (0, 0)),
            pl.BlockSpec((HD, D), lambda i: (0, 0)),
            pl.BlockSpec((1, D), lambda i: (0, 0)),
            pl.BlockSpec((D, D), lambda i: (0, 0)),
            pl.BlockSpec((1, D), lambda i: (0, 0)),
            pl.BlockSpec((BLK, 1), lambda i: (i, 0)),
        ],
        out_specs=[
            pl.BlockSpec((BLK, D), lambda i: (i, 0)),
            pl.BlockSpec((B, D), lambda i: (0, 0)),
        ],
        out_shape=[
            jax.ShapeDtypeStruct((NP, D), _F32),
            jax.ShapeDtypeStruct((B, D), _F32),
        ],
    )(a0, a1, wa, wb, b, gpw, gpb, gid_col)


def _sigmoid(x):
    return 1.0 / (1.0 + jnp.exp(-x))


def _heads_body(hg, fanw1, fanb1, fanw2, fanb2, fiw1, fib1, fiw2, fib2,
                faea, faec, faeb1, faew2, faeb2,
                pnode_ref, hv_ref, pedge_ref):
    g = hg[...]
    t = _sigmoid(_dot(g, fanw1[...]) + fanb1[...])
    logits = _dot(t, fanw2[...]) + fanb2[...]
    m = jnp.max(logits, axis=1, keepdims=True)
    e = jnp.exp(logits - m)
    pnode_ref[...] = e / jnp.sum(e, axis=1, keepdims=True)
    t = _sigmoid(_dot(g, fiw1[...]) + fib1[...])
    hv = _dot(t, fiw2[...]) + fib2[...]
    hv_ref[...] = hv
    t = _sigmoid(_dot(g, faea[...]) + _dot(hv, faec[...]) + faeb1[...])
    pedge_ref[...] = _sigmoid(_dot(t, faew2[...]) + faeb2[...])


def _heads(hg, p):
    full = lambda s: pl.BlockSpec(s, lambda: tuple([0] * len(s)))
    fan, fi, fae = p["fan"], p["finit"], p["fae"]
    args = [
        hg,
        fan["W1"], fan["b1"][None, :], fan["W2"], fan["b2"][None, :],
        fi["W1"], fi["b1"][None, :], fi["W2"], fi["b2"][None, :],
        fae["W1"][:D], fae["W1"][D:], fae["b1"][None, :],
        fae["W2"], fae["b2"][None, :],
    ]
    return pl.pallas_call(
        _heads_body,
        in_specs=[full(a.shape) for a in args],
        out_specs=[full((B, 2)), full((B, D)), full((B, 1))],
        out_shape=[
            jax.ShapeDtypeStruct((B, 2), _F32),
            jax.ShapeDtypeStruct((B, D), _F32),
            jax.ShapeDtypeStruct((B, 1), _F32),
        ],
    )(*args)


def _score_body(h, hv, gid, fsa, fsc, fsb1, fsw2, fsb2, sc_ref):
    iota = lax.broadcasted_iota(jnp.int32, (BLK, B), 1)
    mask = (gid[...] == iota).astype(_F32)
    hvn = _dot(mask, hv[...])
    t = _sigmoid(_dot(h[...], fsa[...]) + _dot(hvn, fsc[...]) + fsb1[...])
    sc_ref[...] = _dot(t, fsw2[...]) + fsb2[...]


def _score(h, hv, gid_col, p):
    fs = p["fs"]
    return pl.pallas_call(
        _score_body,
        grid=(NP // BLK,),
        in_specs=[
            pl.BlockSpec((BLK, D), lambda i: (i, 0)),
            pl.BlockSpec((B, D), lambda i: (0, 0)),
            pl.BlockSpec((BLK, 1), lambda i: (i, 0)),
            pl.BlockSpec((D, 2 * D), lambda i: (0, 0)),
            pl.BlockSpec((D, 2 * D), lambda i: (0, 0)),
            pl.BlockSpec((1, 2 * D), lambda i: (0, 0)),
            pl.BlockSpec((2 * D, 1), lambda i: (0, 0)),
            pl.BlockSpec((1, 1), lambda i: (0, 0)),
        ],
        out_specs=pl.BlockSpec((BLK, 1), lambda i: (i, 0)),
        out_shape=jax.ShapeDtypeStruct((NP, 1), _F32),
    )(h, hv, gid_col, fs["W1"][:D], fs["W1"][D:], fs["b1"][None, :],
      fs["W2"], fs["b2"][None, :])


def _softmax_body(sc_ref, gid_ref, s_ref):
    sc = sc_ref[...]
    iota = lax.broadcasted_iota(jnp.int32, (NP, B), 1)
    maskb = gid_ref[...] == iota
    mask = maskb.astype(_F32)
    m = jnp.max(jnp.where(maskb, sc, -1e30), axis=0)
    mrow = _dot(mask, m[None, :], dims=((1,), (1,)))
    e = jnp.exp(sc - mrow)
    z = _dot(e, mask, dims=((0,), (0,)))
    zrow = _dot(mask, z, dims=((1,), (1,)))
    s_ref[...] = e / zrow


def _softmax(score, gid_col):
    full = lambda s: pl.BlockSpec(s, lambda: (0, 0))
    return pl.pallas_call(
        _softmax_body,
        in_specs=[full((NP, 1)), full((NP, 1))],
        out_specs=full((NP, 1)),
        out_shape=jax.ShapeDtypeStruct((NP, 1), _F32),
    )(score, gid_col)


def kernel(x, edge_index, graph_ids, params):
    pad_e = EP - E
    pad_n = NP - N
    src = jnp.concatenate(
        [edge_index[0], (jnp.arange(pad_e, dtype=jnp.int32) * 97) % N])
    dst = jnp.concatenate(
        [edge_index[1], N + (jnp.arange(pad_e, dtype=jnp.int32) % pad_n)])
    src2d = src.reshape(EP // 128, 128)
    dst2d = dst.reshape(EP // 128, 128)

    xp = jnp.concatenate([x, jnp.zeros((pad_n, D), _F32)], axis=0)
    gid_col = jnp.concatenate(
        [graph_ids, jnp.full((pad_n,), B, jnp.int32)])[:, None]
    zeros = jnp.zeros((NP, HD), _F32)

    a0, a1 = _sc_agg(xp[:, :HD], xp[:, HD:], src2d, dst2d, zeros)
    w0 = params["gcn_W"][0]
    h0, h1 = _round1(a0, a1, w0[:HD], w0[HD:], params["gcn_b"][0][None, :])

    a0, a1 = _sc_agg(h0, h1, src2d, dst2d, zeros)
    w1 = params["gcn_W"][1]
    h, hg = _round2(a0, a1, w1[:HD], w1[HD:], params["gcn_b"][1][None, :],
                    params["gp_W"], params["gp_b"][None, :], gid_col)

    p_node, hv, p_edge = _heads(hg, params)
    score = _score(h, hv, gid_col, params)
    s = _softmax(score, gid_col)
    return (p_node, p_edge, s[:N, 0])


# speedup vs baseline: 4.4366x; 4.4366x over previous
import functools

import jax
import jax.numpy as jnp
from jax import lax
from jax.experimental import pallas as pl
from jax.experimental.pallas import tpu as pltpu
from jax.experimental.pallas import tpu_sc as plsc

N = 10000
E = 160000
D = 256
B = 16
HD = 128
NP = 10240
EP = 163840
NS = 16
EPT = EP // NS
CH = 256
NCH = EPT // CH
RPT = NP // NS
BLK = 1024

_F32 = jnp.float32
_HIGH = lax.Precision.HIGHEST


def _dot(a, b, dims=None):
    if dims is None:
        return jnp.dot(a, b, preferred_element_type=_F32, precision=_HIGH)
    return lax.dot_general(a, b, (dims, ((), ())),
                           preferred_element_type=_F32, precision=_HIGH)


def _sc_agg_kernel(h0_hbm, h1_hbm, src_hbm, dst_hbm, z_hbm, o0_hbm, o1_hbm,
                   sidx, didx, rows, acc, sem):
    cid = lax.axis_index("c")
    sid = lax.axis_index("s")

    pltpu.sync_copy(z_hbm.at[pl.ds(sid * RPT, RPT)],
                    acc.at[pl.ds(sid * RPT, RPT)])
    plsc.subcore_barrier()

    def run(h_hbm, out_hbm):
        def chunk(k, carry):
            rbase = sid * (EPT // 128) + k * (CH // 128)
            pltpu.sync_copy(src_hbm.at[pl.ds(rbase, CH // 128)], sidx)
            pltpu.sync_copy(dst_hbm.at[pl.ds(rbase, CH // 128)], didx)
            cps = [
                pltpu.async_copy(h_hbm.at[sidx.at[j]],
                                 rows.at[pl.ds(j * 128, 128)], sem)
                for j in range(CH // 128)
            ]
            for cp in cps:
                cp.wait()
            for j in range(CH // 128):
                pltpu.sync_copy(rows.at[pl.ds(j * 128, 128)],
                                acc.at[didx.at[j]], add=True)
            return carry

        lax.fori_loop(0, NCH, chunk, 0)
        plsc.subcore_barrier()
        pltpu.sync_copy(acc.at[pl.ds(sid * RPT, RPT)],
                        out_hbm.at[pl.ds(sid * RPT, RPT)])

    @pl.when(cid == 0)
    def _():
        run(h0_hbm, o0_hbm)

    @pl.when(cid == 1)
    def _():
        run(h1_hbm, o1_hbm)


def _sc_agg(h0, h1, src2d, dst2d, zeros):
    mesh = plsc.VectorSubcoreMesh(core_axis_name="c", subcore_axis_name="s")
    k = functools.partial(
        pl.kernel, mesh=mesh,
        out_type=[jax.ShapeDtypeStruct((NP, HD), _F32),
                  jax.ShapeDtypeStruct((NP, HD), _F32)],
        scratch_types=[
            pltpu.VMEM((CH // 128, 128), jnp.int32),
            pltpu.VMEM((CH // 128, 128), jnp.int32),
            pltpu.VMEM((CH, HD), _F32),
            pltpu.VMEM_SHARED((NP, HD), _F32),
            pltpu.SemaphoreType.DMA,
        ],
    )(_sc_agg_kernel)
    return k(h0, h1, src2d, dst2d, zeros)


def _round1_body(a0, a1, wa, wb, b, o0, o1):
    h = _dot(a0[...], wa[...]) + _dot(a1[...], wb[...]) + b[...]
    h = jnp.maximum(h, 0.0)
    o0[...] = h[:, :HD]
    o1[...] = h[:, HD:]


def _round1(a0, a1, wa, wb, b):
    return pl.pallas_call(
        _round1_body,
        grid=(NP // BLK,),
        in_specs=[
            pl.BlockSpec((BLK, HD), lambda i: (i, 0)),
            pl.BlockSpec((BLK, HD), lambda i: (i, 0)),
            pl.BlockSpec((HD, D), lambda i: (0, 0)),
            pl.BlockSpec((HD, D), lambda i: (0, 0)),
            pl.BlockSpec((1, D), lambda i: (0, 0)),
        ],
        out_specs=[pl.BlockSpec((BLK, HD), lambda i: (i, 0))] * 2,
        out_shape=[jax.ShapeDtypeStruct((NP, HD), _F32)] * 2,
    )(a0, a1, wa, wb, b)


def _round2_body(a0, a1, wa, wb, b, gpw, gpb, gid, h_ref, hg_ref):
    i = pl.program_id(0)
    h = _dot(a0[...], wa[...]) + _dot(a1[...], wb[...]) + b[...]
    h = jnp.maximum(h, 0.0)
    h_ref[...] = h
    y = _dot(h, gpw[...]) + gpb[...]
    iota = lax.broadcasted_iota(jnp.int32, (BLK, B), 1)
    mask = (gid[...] == iota).astype(_F32)
    part = _dot(mask, y, dims=((0,), (0,)))

    @pl.when(i == 0)
    def _():
        hg_ref[...] = part

    @pl.when(i > 0)
    def _():
        hg_ref[...] += part


def _round2(a0, a1, wa, wb, b, gpw, gpb, gid_col):
    return pl.pallas_call(
        _round2_body,
        grid=(NP // BLK,),
        in_specs=[
            pl.BlockSpec((BLK, HD), lambda i: (i, 0)),
            pl.BlockSpec((BLK, HD), lambda i: (i, 0)),
            pl.BlockSpec((HD, D), lambda i: (0, 0)),
            pl.BlockSpec((HD, D), lambda i: (0, 0)),
            pl.BlockSpec((1, D), lambda i: (0, 0)),
            pl.BlockSpec((D, D), lambda i: (0, 0)),
            pl.BlockSpec((1, D), lambda i: (0, 0)),
            pl.BlockSpec((BLK, 1), lambda i: (i, 0)),
        ],
        out_specs=[
            pl.BlockSpec((BLK, D), lambda i: (i, 0)),
            pl.BlockSpec((B, D), lambda i: (0, 0)),
        ],
        out_shape=[
            jax.ShapeDtypeStruct((NP, D), _F32),
            jax.ShapeDtypeStruct((B, D), _F32),
        ],
    )(a0, a1, wa, wb, b, gpw, gpb, gid_col)


def _sigmoid(x):
    return 1.0 / (1.0 + jnp.exp(-x))


def _heads_body(hg, fanw1, fanb1, fanw2, fanb2, fiw1, fib1, fiw2, fib2,
                faea, faec, faeb1, faew2, faeb2,
                pnode_ref, hv_ref, pedge_ref):
    g = hg[...]
    t = _sigmoid(_dot(g, fanw1[...]) + fanb1[...])
    logits = _dot(t, fanw2[...]) + fanb2[...]
    m = jnp.max(logits, axis=1, keepdims=True)
    e = jnp.exp(logits - m)
    pnode_ref[...] = e / jnp.sum(e, axis=1, keepdims=True)
    t = _sigmoid(_dot(g, fiw1[...]) + fib1[...])
    hv = _dot(t, fiw2[...]) + fib2[...]
    hv_ref[...] = hv
    t = _sigmoid(_dot(g, faea[...]) + _dot(hv, faec[...]) + faeb1[...])
    pedge_ref[...] = _sigmoid(_dot(t, faew2[...]) + faeb2[...])


def _heads(hg, p):
    full = lambda s: pl.BlockSpec(s, lambda: tuple([0] * len(s)))
    fan, fi, fae = p["fan"], p["finit"], p["fae"]
    args = [
        hg,
        fan["W1"], fan["b1"][None, :], fan["W2"], fan["b2"][None, :],
        fi["W1"], fi["b1"][None, :], fi["W2"], fi["b2"][None, :],
        fae["W1"][:D], fae["W1"][D:], fae["b1"][None, :],
        fae["W2"], fae["b2"][None, :],
    ]
    return pl.pallas_call(
        _heads_body,
        in_specs=[full(a.shape) for a in args],
        out_specs=[full((B, 2)), full((B, D)), full((B, 1))],
        out_shape=[
            jax.ShapeDtypeStruct((B, 2), _F32),
            jax.ShapeDtypeStruct((B, D), _F32),
            jax.ShapeDtypeStruct((B, 1), _F32),
        ],
    )(*args)


def _score_body(h, hv, gid, fsa, fsc, fsb1, fsw2, fsb2, sc_ref):
    iota = lax.broadcasted_iota(jnp.int32, (BLK, B), 1)
    mask = (gid[...] == iota).astype(_F32)
    hvn = _dot(mask, hv[...])
    t = _sigmoid(_dot(h[...], fsa[...]) + _dot(hvn, fsc[...]) + fsb1[...])
    sc_ref[...] = _dot(t, fsw2[...]) + fsb2[...]


def _score(h, hv, gid_col, p):
    fs = p["fs"]
    return pl.pallas_call(
        _score_body,
        grid=(NP // BLK,),
        in_specs=[
            pl.BlockSpec((BLK, D), lambda i: (i, 0)),
            pl.BlockSpec((B, D), lambda i: (0, 0)),
            pl.BlockSpec((BLK, 1), lambda i: (i, 0)),
            pl.BlockSpec((D, 2 * D), lambda i: (0, 0)),
            pl.BlockSpec((D, 2 * D), lambda i: (0, 0)),
            pl.BlockSpec((1, 2 * D), lambda i: (0, 0)),
            pl.BlockSpec((2 * D, 1), lambda i: (0, 0)),
            pl.BlockSpec((1, 1), lambda i: (0, 0)),
        ],
        out_specs=pl.BlockSpec((BLK, 1), lambda i: (i, 0)),
        out_shape=jax.ShapeDtypeStruct((NP, 1), _F32),
    )(h, hv, gid_col, fs["W1"][:D], fs["W1"][D:], fs["b1"][None, :],
      fs["W2"], fs["b2"][None, :])


def _softmax_body(sc_ref, gid_ref, s_ref):
    sc = sc_ref[...]
    iota = lax.broadcasted_iota(jnp.int32, (NP, B), 1)
    maskb = gid_ref[...] == iota
    mask = maskb.astype(_F32)
    m = jnp.max(jnp.where(maskb, sc, -1e30), axis=0)
    mrow = _dot(mask, m[None, :], dims=((1,), (1,)))
    e = jnp.exp(sc - mrow)
    z = _dot(e, mask, dims=((0,), (0,)))
    zrow = _dot(mask, z, dims=((1,), (1,)))
    s_ref[...] = e / zrow


def _softmax(score, gid_col):
    full = lambda s: pl.BlockSpec(s, lambda: (0, 0))
    return pl.pallas_call(
        _softmax_body,
        in_specs=[full((NP, 1)), full((NP, 1))],
        out_specs=full((NP, 1)),
        out_shape=jax.ShapeDtypeStruct((NP, 1), _F32),
    )(score, gid_col)


def kernel(x, edge_index, graph_ids, params):
    pad_e = EP - E
    pad_n = NP - N
    src = jnp.concatenate(
        [edge_index[0], (jnp.arange(pad_e, dtype=jnp.int32) * 97) % N])
    dst = jnp.concatenate(
        [edge_index[1], N + (jnp.arange(pad_e, dtype=jnp.int32) % pad_n)])
    src2d = src.reshape(EP // 128, 128)
    dst2d = dst.reshape(EP // 128, 128)

    xp = jnp.concatenate([x, jnp.zeros((pad_n, D), _F32)], axis=0)
    gid_col = jnp.concatenate(
        [graph_ids, jnp.full((pad_n,), B, jnp.int32)])[:, None]
    zeros = jnp.zeros((NP, HD), _F32)

    a0, a1 = _sc_agg(xp[:, :HD], xp[:, HD:], src2d, dst2d, zeros)
    w0 = params["gcn_W"][0]
    h0, h1 = _round1(a0, a1, w0[:HD], w0[HD:], params["gcn_b"][0][None, :])

    a0, a1 = _sc_agg(h0, h1, src2d, dst2d, zeros)
    w1 = params["gcn_W"][1]
    h, hg = _round2(a0, a1, w1[:HD], w1[HD:], params["gcn_b"][1][None, :],
                    params["gp_W"], params["gp_b"][None, :], gid_col)

    p_node, hv, p_edge = _heads(hg, params)
    score = _score(h, hv, gid_col, params)
    s = _softmax(score, gid_col)
    return (p_node, p_edge, s[:N, 0])


# speedup vs baseline: 5.5081x; 1.2415x over previous
import functools

import jax
import jax.numpy as jnp
from jax import lax
from jax.experimental import pallas as pl
from jax.experimental.pallas import tpu as pltpu
from jax.experimental.pallas import tpu_sc as plsc

N = 10000
E = 160000
D = 256
B = 16
HD = 128
NP = 10240
EP = 163840
NS = 16
EPT = EP // NS
CH = 128
NCH = EPT // CH
RPT = NP // NS
BLK = 1024

_F32 = jnp.float32
_HIGH = lax.Precision.HIGHEST


def _dot(a, b, dims=None):
    if dims is None:
        return jnp.dot(a, b, preferred_element_type=_F32, precision=_HIGH)
    return lax.dot_general(a, b, (dims, ((), ())),
                           preferred_element_type=_F32, precision=_HIGH)


def _sc_agg_kernel(h0_hbm, h1_hbm, sd_hbm, z_hbm, o0_hbm, o1_hbm,
                   idx0, idx1, idx2, idx3, rows0, rows1, acc,
                   is0, is1, is2, is3, gs0, gs1, ss0, ss1):
    cid = lax.axis_index("c")
    sid = lax.axis_index("s")
    idxs = (idx0, idx1, idx2, idx3)
    isems = (is0, is1, is2, is3)
    rows = (rows0, rows1)
    gsems = (gs0, gs1)
    ssems = (ss0, ss1)

    pltpu.sync_copy(z_hbm.at[pl.ds(sid * RPT, RPT)],
                    acc.at[pl.ds(sid * RPT, RPT)])
    plsc.subcore_barrier()

    base = sid * NCH

    def run(h_hbm, out_hbm):
        dummy = h_hbm.at[pl.ds(0, CH)]

        def fetch_idx(k, slot):
            pltpu.async_copy(sd_hbm.at[base + k], idxs[slot], isems[slot])

        def wait_idx(slot):
            pltpu.make_async_copy(sd_hbm.at[0], idxs[slot], isems[slot]).wait()

        def start_gather(slot, islot):
            pltpu.async_copy(h_hbm.at[idxs[islot].at[0]], rows[slot],
                             gsems[slot])

        def wait_gather(slot):
            pltpu.make_async_copy(dummy, rows[slot], gsems[slot]).wait()

        def start_scatter(slot, islot):
            pltpu.async_copy(rows[slot], acc.at[idxs[islot].at[1]],
                             ssems[slot], add=True)

        def wait_scatter(slot):
            pltpu.make_async_copy(dummy, rows[slot], ssems[slot]).wait()

        fetch_idx(0, 0)
        fetch_idx(1, 1)
        wait_idx(0)
        start_gather(0, 0)

        def body(g, carry):
            for u in range(4):
                k = 4 * g + u
                b = u % 2
                o = 1 - b

                wait_gather(b)
                start_scatter(b, u)

                @pl.when(k >= 1)
                def _():
                    wait_scatter(o)

                @pl.when(k + 2 < NCH)
                def _():
                    fetch_idx(k + 2, (u + 2) % 4)

                @pl.when(k + 1 < NCH)
                def _():
                    wait_idx((u + 1) % 4)
                    start_gather(o, (u + 1) % 4)

            return carry

        lax.fori_loop(0, NCH // 4, body, 0)
        wait_scatter((NCH - 1) % 2)
        plsc.subcore_barrier()
        pltpu.sync_copy(acc.at[pl.ds(sid * RPT, RPT)],
                        out_hbm.at[pl.ds(sid * RPT, RPT)])

    @pl.when(cid == 0)
    def _():
        run(h0_hbm, o0_hbm)

    @pl.when(cid == 1)
    def _():
        run(h1_hbm, o1_hbm)


def _sc_agg(h0, h1, sd3d, zeros):
    mesh = plsc.VectorSubcoreMesh(core_axis_name="c", subcore_axis_name="s")
    k = functools.partial(
        pl.kernel, mesh=mesh,
        out_type=[jax.ShapeDtypeStruct((NP, HD), _F32),
                  jax.ShapeDtypeStruct((NP, HD), _F32)],
        scratch_types=(
            [pltpu.VMEM((2, 128), jnp.int32)] * 4
            + [pltpu.VMEM((CH, HD), _F32)] * 2
            + [pltpu.VMEM_SHARED((NP, HD), _F32)]
            + [pltpu.SemaphoreType.DMA] * 8
        ),
    )(_sc_agg_kernel)
    return k(h0, h1, sd3d, zeros)


def _round1_body(a0, a1, wa, wb, b, o0, o1):
    h = _dot(a0[...], wa[...]) + _dot(a1[...], wb[...]) + b[...]
    h = jnp.maximum(h, 0.0)
    o0[...] = h[:, :HD]
    o1[...] = h[:, HD:]


def _round1(a0, a1, wa, wb, b):
    return pl.pallas_call(
        _round1_body,
        grid=(NP // BLK,),
        in_specs=[
            pl.BlockSpec((BLK, HD), lambda i: (i, 0)),
            pl.BlockSpec((BLK, HD), lambda i: (i, 0)),
            pl.BlockSpec((HD, D), lambda i: (0, 0)),
            pl.BlockSpec((HD, D), lambda i: (0, 0)),
            pl.BlockSpec((1, D), lambda i: (0, 0)),
        ],
        out_specs=[pl.BlockSpec((BLK, HD), lambda i: (i, 0))] * 2,
        out_shape=[jax.ShapeDtypeStruct((NP, HD), _F32)] * 2,
    )(a0, a1, wa, wb, b)


def _round2_body(a0, a1, wa, wb, b, gpw, gpb, gid, h_ref, hg_ref):
    i = pl.program_id(0)
    h = _dot(a0[...], wa[...]) + _dot(a1[...], wb[...]) + b[...]
    h = jnp.maximum(h, 0.0)
    h_ref[...] = h
    y = _dot(h, gpw[...]) + gpb[...]
    iota = lax.broadcasted_iota(jnp.int32, (BLK, B), 1)
    mask = (gid[...] == iota).astype(_F32)
    part = _dot(mask, y, dims=((0,), (0,)))

    @pl.when(i == 0)
    def _():
        hg_ref[...] = part

    @pl.when(i > 0)
    def _():
        hg_ref[...] += part


def _round2(a0, a1, wa, wb, b, gpw, gpb, gid_col):
    return pl.pallas_call(
        _round2_body,
        grid=(NP // BLK,),
        in_specs=[
            pl.BlockSpec((BLK, HD), lambda i: (i, 0)),
            pl.BlockSpec((BLK, HD), lambda i: (i, 0)),
            pl.BlockSpec((HD, D), lambda i: (0, 0)),
            pl.BlockSpec((HD, D), lambda i: (0, 0)),
            pl.BlockSpec((1, D), lambda i: (0, 0)),
            pl.BlockSpec((D, D), lambda i: (0, 0)),
            pl.BlockSpec((1, D), lambda i: (0, 0)),
            pl.BlockSpec((BLK, 1), lambda i: (i, 0)),
        ],
        out_specs=[
            pl.BlockSpec((BLK, D), lambda i: (i, 0)),
            pl.BlockSpec((B, D), lambda i: (0, 0)),
        ],
        out_shape=[
            jax.ShapeDtypeStruct((NP, D), _F32),
            jax.ShapeDtypeStruct((B, D), _F32),
        ],
    )(a0, a1, wa, wb, b, gpw, gpb, gid_col)


def _sigmoid(x):
    return 1.0 / (1.0 + jnp.exp(-x))


def _heads_body(hg, fanw1, fanb1, fanw2, fanb2, fiw1, fib1, fiw2, fib2,
                faea, faec, faeb1, faew2, faeb2,
                pnode_ref, hv_ref, pedge_ref):
    g = hg[...]
    t = _sigmoid(_dot(g, fanw1[...]) + fanb1[...])
    logits = _dot(t, fanw2[...]) + fanb2[...]
    m = jnp.max(logits, axis=1, keepdims=True)
    e = jnp.exp(logits - m)
    pnode_ref[...] = e / jnp.sum(e, axis=1, keepdims=True)
    t = _sigmoid(_dot(g, fiw1[...]) + fib1[...])
    hv = _dot(t, fiw2[...]) + fib2[...]
    hv_ref[...] = hv
    t = _sigmoid(_dot(g, faea[...]) + _dot(hv, faec[...]) + faeb1[...])
    pedge_ref[...] = _sigmoid(_dot(t, faew2[...]) + faeb2[...])


def _heads(hg, p):
    full = lambda s: pl.BlockSpec(s, lambda: tuple([0] * len(s)))
    fan, fi, fae = p["fan"], p["finit"], p["fae"]
    args = [
        hg,
        fan["W1"], fan["b1"][None, :], fan["W2"], fan["b2"][None, :],
        fi["W1"], fi["b1"][None, :], fi["W2"], fi["b2"][None, :],
        fae["W1"][:D], fae["W1"][D:], fae["b1"][None, :],
        fae["W2"], fae["b2"][None, :],
    ]
    return pl.pallas_call(
        _heads_body,
        in_specs=[full(a.shape) for a in args],
        out_specs=[full((B, 2)), full((B, D)), full((B, 1))],
        out_shape=[
            jax.ShapeDtypeStruct((B, 2), _F32),
            jax.ShapeDtypeStruct((B, D), _F32),
            jax.ShapeDtypeStruct((B, 1), _F32),
        ],
    )(*args)


def _score_body(h, hv, gid, fsa, fsc, fsb1, fsw2, fsb2, sc_ref):
    iota = lax.broadcasted_iota(jnp.int32, (BLK, B), 1)
    mask = (gid[...] == iota).astype(_F32)
    hvn = _dot(mask, hv[...])
    t = _sigmoid(_dot(h[...], fsa[...]) + _dot(hvn, fsc[...]) + fsb1[...])
    sc_ref[...] = _dot(t, fsw2[...]) + fsb2[...]


def _score(h, hv, gid_col, p):
    fs = p["fs"]
    return pl.pallas_call(
        _score_body,
        grid=(NP // BLK,),
        in_specs=[
            pl.BlockSpec((BLK, D), lambda i: (i, 0)),
            pl.BlockSpec((B, D), lambda i: (0, 0)),
            pl.BlockSpec((BLK, 1), lambda i: (i, 0)),
            pl.BlockSpec((D, 2 * D), lambda i: (0, 0)),
            pl.BlockSpec((D, 2 * D), lambda i: (0, 0)),
            pl.BlockSpec((1, 2 * D), lambda i: (0, 0)),
            pl.BlockSpec((2 * D, 1), lambda i: (0, 0)),
            pl.BlockSpec((1, 1), lambda i: (0, 0)),
        ],
        out_specs=pl.BlockSpec((BLK, 1), lambda i: (i, 0)),
        out_shape=jax.ShapeDtypeStruct((NP, 1), _F32),
    )(h, hv, gid_col, fs["W1"][:D], fs["W1"][D:], fs["b1"][None, :],
      fs["W2"], fs["b2"][None, :])


def _softmax_body(sc_ref, gid_ref, s_ref):
    sc = sc_ref[...]
    iota = lax.broadcasted_iota(jnp.int32, (NP, B), 1)
    maskb = gid_ref[...] == iota
    mask = maskb.astype(_F32)
    m = jnp.max(jnp.where(maskb, sc, -1e30), axis=0)
    mrow = _dot(mask, m[None, :], dims=((1,), (1,)))
    e = jnp.exp(sc - mrow)
    z = _dot(e, mask, dims=((0,), (0,)))
    zrow = _dot(mask, z, dims=((1,), (1,)))
    s_ref[...] = e / zrow


def _softmax(score, gid_col):
    full = lambda s: pl.BlockSpec(s, lambda: (0, 0))
    return pl.pallas_call(
        _softmax_body,
        in_specs=[full((NP, 1)), full((NP, 1))],
        out_specs=full((NP, 1)),
        out_shape=jax.ShapeDtypeStruct((NP, 1), _F32),
    )(score, gid_col)


def kernel(x, edge_index, graph_ids, params):
    pad_e = EP - E
    pad_n = NP - N
    src = jnp.concatenate(
        [edge_index[0], (jnp.arange(pad_e, dtype=jnp.int32) * 97) % N])
    dst = jnp.concatenate(
        [edge_index[1], N + (jnp.arange(pad_e, dtype=jnp.int32) % pad_n)])
    sd3d = jnp.stack(
        [src.reshape(EP // 128, 128), dst.reshape(EP // 128, 128)], axis=1)

    xp = jnp.concatenate([x, jnp.zeros((pad_n, D), _F32)], axis=0)
    gid_col = jnp.concatenate(
        [graph_ids, jnp.full((pad_n,), B, jnp.int32)])[:, None]
    zeros = jnp.zeros((NP, HD), _F32)

    a0, a1 = _sc_agg(xp[:, :HD], xp[:, HD:], sd3d, zeros)
    w0 = params["gcn_W"][0]
    h0, h1 = _round1(a0, a1, w0[:HD], w0[HD:], params["gcn_b"][0][None, :])

    a0, a1 = _sc_agg(h0, h1, sd3d, zeros)
    w1 = params["gcn_W"][1]
    h, hg = _round2(a0, a1, w1[:HD], w1[HD:], params["gcn_b"][1][None, :],
                    params["gp_W"], params["gp_b"][None, :], gid_col)

    p_node, hv, p_edge = _heads(hg, params)
    score = _score(h, hv, gid_col, params)
    s = _softmax(score, gid_col)
    return (p_node, p_edge, s[:N, 0])


# speedup vs baseline: 7.1795x; 1.3034x over previous
import functools

import jax
import jax.numpy as jnp
from jax import lax
from jax.experimental import pallas as pl
from jax.experimental.pallas import tpu as pltpu
from jax.experimental.pallas import tpu_sc as plsc

N = 10000
E = 160000
D = 256
B = 16
HD = 128
NP = 10240
EP = 163840
NS = 16
EPT = EP // NS
CH = 128
NCH = EPT // CH
RPT = NP // NS
BLK = 1024

_F32 = jnp.float32
_HIGH = lax.Precision.DEFAULT


def _dot(a, b, dims=None):
    if dims is None:
        return jnp.dot(a, b, preferred_element_type=_F32, precision=_HIGH)
    return lax.dot_general(a, b, (dims, ((), ())),
                           preferred_element_type=_F32, precision=_HIGH)


def _sc_agg_kernel(h0_hbm, h1_hbm, sd_hbm, z_hbm, o0_hbm, o1_hbm,
                   idx0, idx1, idx2, idx3, rows0, rows1, acc,
                   is0, is1, is2, is3, gs0, gs1, ss0, ss1):
    cid = lax.axis_index("c")
    sid = lax.axis_index("s")
    idxs = (idx0, idx1, idx2, idx3)
    isems = (is0, is1, is2, is3)
    rows = (rows0, rows1)
    gsems = (gs0, gs1)
    ssems = (ss0, ss1)

    pltpu.sync_copy(z_hbm.at[pl.ds(sid * RPT, RPT)],
                    acc.at[pl.ds(sid * RPT, RPT)])
    plsc.subcore_barrier()

    base = sid * NCH

    def run(h_hbm, out_hbm):
        dummy = h_hbm.at[pl.ds(0, CH)]

        def fetch_idx(k, slot):
            pltpu.async_copy(sd_hbm.at[base + k], idxs[slot], isems[slot])

        def wait_idx(slot):
            pltpu.make_async_copy(sd_hbm.at[0], idxs[slot], isems[slot]).wait()

        def start_gather(slot, islot):
            pltpu.async_copy(h_hbm.at[idxs[islot].at[0]], rows[slot],
                             gsems[slot])

        def wait_gather(slot):
            pltpu.make_async_copy(dummy, rows[slot], gsems[slot]).wait()

        def start_scatter(slot, islot):
            pltpu.async_copy(rows[slot], acc.at[idxs[islot].at[1]],
                             ssems[slot], add=True)

        def wait_scatter(slot):
            pltpu.make_async_copy(dummy, rows[slot], ssems[slot]).wait()

        fetch_idx(0, 0)
        fetch_idx(1, 1)
        wait_idx(0)
        start_gather(0, 0)

        def body(g, carry):
            for u in range(4):
                k = 4 * g + u
                b = u % 2
                o = 1 - b

                wait_gather(b)
                start_scatter(b, u)

                @pl.when(k >= 1)
                def _():
                    wait_scatter(o)

                @pl.when(k + 2 < NCH)
                def _():
                    fetch_idx(k + 2, (u + 2) % 4)

                @pl.when(k + 1 < NCH)
                def _():
                    wait_idx((u + 1) % 4)
                    start_gather(o, (u + 1) % 4)

            return carry

        lax.fori_loop(0, NCH // 4, body, 0)
        wait_scatter((NCH - 1) % 2)
        plsc.subcore_barrier()
        pltpu.sync_copy(acc.at[pl.ds(sid * RPT, RPT)],
                        out_hbm.at[pl.ds(sid * RPT, RPT)])

    @pl.when(cid == 0)
    def _():
        run(h0_hbm, o0_hbm)

    @pl.when(cid == 1)
    def _():
        run(h1_hbm, o1_hbm)


def _sc_agg(h0, h1, sd3d, zeros):
    mesh = plsc.VectorSubcoreMesh(core_axis_name="c", subcore_axis_name="s")
    k = functools.partial(
        pl.kernel, mesh=mesh,
        out_type=[jax.ShapeDtypeStruct((NP, HD), _F32),
                  jax.ShapeDtypeStruct((NP, HD), _F32)],
        scratch_types=(
            [pltpu.VMEM((2, 128), jnp.int32)] * 4
            + [pltpu.VMEM((CH, HD), _F32)] * 2
            + [pltpu.VMEM_SHARED((NP, HD), _F32)]
            + [pltpu.SemaphoreType.DMA] * 8
        ),
    )(_sc_agg_kernel)
    return k(h0, h1, sd3d, zeros)


def _round1_body(a0, a1, wa, wb, b, o0, o1):
    h = _dot(a0[...], wa[...]) + _dot(a1[...], wb[...]) + b[...]
    h = jnp.maximum(h, 0.0)
    o0[...] = h[:, :HD]
    o1[...] = h[:, HD:]


def _round1(a0, a1, wa, wb, b):
    return pl.pallas_call(
        _round1_body,
        grid=(NP // BLK,),
        in_specs=[
            pl.BlockSpec((BLK, HD), lambda i: (i, 0)),
            pl.BlockSpec((BLK, HD), lambda i: (i, 0)),
            pl.BlockSpec((HD, D), lambda i: (0, 0)),
            pl.BlockSpec((HD, D), lambda i: (0, 0)),
            pl.BlockSpec((1, D), lambda i: (0, 0)),
        ],
        out_specs=[pl.BlockSpec((BLK, HD), lambda i: (i, 0))] * 2,
        out_shape=[jax.ShapeDtypeStruct((NP, HD), _F32)] * 2,
    )(a0, a1, wa, wb, b)


def _round2_body(a0, a1, wa, wb, b, gpw, gpb, gid, h_ref, hg_ref):
    i = pl.program_id(0)
    h = _dot(a0[...], wa[...]) + _dot(a1[...], wb[...]) + b[...]
    h = jnp.maximum(h, 0.0)
    h_ref[...] = h
    y = _dot(h, gpw[...]) + gpb[...]
    iota = lax.broadcasted_iota(jnp.int32, (BLK, B), 1)
    mask = (gid[...] == iota).astype(_F32)
    part = _dot(mask, y, dims=((0,), (0,)))

    @pl.when(i == 0)
    def _():
        hg_ref[...] = part

    @pl.when(i > 0)
    def _():
        hg_ref[...] += part


def _round2(a0, a1, wa, wb, b, gpw, gpb, gid_col):
    return pl.pallas_call(
        _round2_body,
        grid=(NP // BLK,),
        in_specs=[
            pl.BlockSpec((BLK, HD), lambda i: (i, 0)),
            pl.BlockSpec((BLK, HD), lambda i: (i, 0)),
            pl.BlockSpec((HD, D), lambda i: (0, 0)),
            pl.BlockSpec((HD, D), lambda i: (0, 0)),
            pl.BlockSpec((1, D), lambda i: (0, 0)),
            pl.BlockSpec((D, D), lambda i: (0, 0)),
            pl.BlockSpec((1, D), lambda i: (0, 0)),
            pl.BlockSpec((BLK, 1), lambda i: (i, 0)),
        ],
        out_specs=[
            pl.BlockSpec((BLK, D), lambda i: (i, 0)),
            pl.BlockSpec((B, D), lambda i: (0, 0)),
        ],
        out_shape=[
            jax.ShapeDtypeStruct((NP, D), _F32),
            jax.ShapeDtypeStruct((B, D), _F32),
        ],
    )(a0, a1, wa, wb, b, gpw, gpb, gid_col)


def _sigmoid(x):
    return 1.0 / (1.0 + jnp.exp(-x))


def _score_body(h, hg, gid,
                fanw1, fanb1, fanw2, fanb2, fiw1, fib1, fiw2, fib2,
                faea, faec, faeb1, faew2, faeb2,
                fsa, fsc, fsb1, fsw2, fsb2,
                sc_ref, pnode_ref, pedge_ref):
    g = hg[...]
    t = _sigmoid(_dot(g, fanw1[...]) + fanb1[...])
    logits = _dot(t, fanw2[...]) + fanb2[...]
    m = jnp.max(logits, axis=1, keepdims=True)
    e = jnp.exp(logits - m)
    pnode_ref[...] = e / jnp.sum(e, axis=1, keepdims=True)
    t = _sigmoid(_dot(g, fiw1[...]) + fib1[...])
    hv = _dot(t, fiw2[...]) + fib2[...]
    t = _sigmoid(_dot(g, faea[...]) + _dot(hv, faec[...]) + faeb1[...])
    pedge_ref[...] = _sigmoid(_dot(t, faew2[...]) + faeb2[...])
    iota = lax.broadcasted_iota(jnp.int32, (BLK, B), 1)
    mask = (gid[...] == iota).astype(_F32)
    hvn = _dot(mask, hv)
    t = _sigmoid(_dot(h[...], fsa[...]) + _dot(hvn, fsc[...]) + fsb1[...])
    sc_ref[...] = _dot(t, fsw2[...]) + fsb2[...]


def _score(h, hg, gid_col, p):
    fan, fi, fae, fs = p["fan"], p["finit"], p["fae"], p["fs"]
    args = [
        h, hg, gid_col,
        fan["W1"], fan["b1"][None, :], fan["W2"], fan["b2"][None, :],
        fi["W1"], fi["b1"][None, :], fi["W2"], fi["b2"][None, :],
        fae["W1"][:D], fae["W1"][D:], fae["b1"][None, :],
        fae["W2"], fae["b2"][None, :],
        fs["W1"][:D], fs["W1"][D:], fs["b1"][None, :],
        fs["W2"], fs["b2"][None, :],
    ]
    in_specs = [
        pl.BlockSpec((BLK, D), lambda i: (i, 0)),
        pl.BlockSpec((B, D), lambda i: (0, 0)),
        pl.BlockSpec((BLK, 1), lambda i: (i, 0)),
    ] + [pl.BlockSpec(a.shape, lambda i: (0, 0)) for a in args[3:]]
    return pl.pallas_call(
        _score_body,
        grid=(NP // BLK,),
        in_specs=in_specs,
        out_specs=[
            pl.BlockSpec((BLK, 1), lambda i: (i, 0)),
            pl.BlockSpec((B, 2), lambda i: (0, 0)),
            pl.BlockSpec((B, 1), lambda i: (0, 0)),
        ],
        out_shape=[
            jax.ShapeDtypeStruct((NP, 1), _F32),
            jax.ShapeDtypeStruct((B, 2), _F32),
            jax.ShapeDtypeStruct((B, 1), _F32),
        ],
    )(*args)


def _softmax_body(sc_ref, gid_ref, s_ref):
    sc = sc_ref[...]
    iota = lax.broadcasted_iota(jnp.int32, (NP, B), 1)
    maskb = gid_ref[...] == iota
    mask = maskb.astype(_F32)
    m = jnp.max(jnp.where(maskb, sc, -1e30), axis=0)
    mrow = _dot(mask, m[None, :], dims=((1,), (1,)))
    e = jnp.exp(sc - mrow)
    z = _dot(e, mask, dims=((0,), (0,)))
    zrow = _dot(mask, z, dims=((1,), (1,)))
    s_ref[...] = e / zrow


def _softmax(score, gid_col):
    full = lambda s: pl.BlockSpec(s, lambda: (0, 0))
    return pl.pallas_call(
        _softmax_body,
        in_specs=[full((NP, 1)), full((NP, 1))],
        out_specs=full((NP, 1)),
        out_shape=jax.ShapeDtypeStruct((NP, 1), _F32),
    )(score, gid_col)


def kernel(x, edge_index, graph_ids, params):
    pad_e = EP - E
    pad_n = NP - N
    src = jnp.concatenate(
        [edge_index[0], (jnp.arange(pad_e, dtype=jnp.int32) * 97) % N])
    dst = jnp.concatenate(
        [edge_index[1], N + (jnp.arange(pad_e, dtype=jnp.int32) % pad_n)])
    sd3d = jnp.stack(
        [src.reshape(EP // 128, 128), dst.reshape(EP // 128, 128)], axis=1)

    gid_col = jnp.concatenate(
        [graph_ids, jnp.full((pad_n,), B, jnp.int32)])[:, None]
    zeros = jnp.zeros((NP, HD), _F32)

    a0, a1 = _sc_agg(x[:, :HD], x[:, HD:], sd3d, zeros)
    w0 = params["gcn_W"][0]
    h0, h1 = _round1(a0, a1, w0[:HD], w0[HD:], params["gcn_b"][0][None, :])

    a0, a1 = _sc_agg(h0, h1, sd3d, zeros)
    w1 = params["gcn_W"][1]
    h, hg = _round2(a0, a1, w1[:HD], w1[HD:], params["gcn_b"][1][None, :],
                    params["gp_W"], params["gp_b"][None, :], gid_col)

    score, p_node, p_edge = _score(h, hg, gid_col, params)
    s = _softmax(score, gid_col)
    return (p_node, p_edge, s[:N, 0])
